# baseline (speedup 1.0000x reference)
import functools
import math

import jax
import jax.numpy as jnp
import numpy as np
from jax import lax
from jax.experimental import pallas as pl
from jax.experimental.pallas import tpu as pltpu
from jax.experimental.pallas import tpu_sc as plsc

VOCAB = 1000000
DIM = 32
MAX_LEN = 200
BATCH = 4096
SEQ = 200

NC, NS = 2, 16
NW = NC * NS
BPW = BATCH // NW
LPC = 4
CHUNKS = SEQ // LPC
DT, DR = DIM // 8, 8
SCALE = math.sqrt(DIM)


def _make_pos_table():
    para = np.arange(MAX_LEN).reshape(-1, 1) / np.power(
        10000.0, np.arange(0, DIM, 2) / DIM)
    pos = np.zeros((MAX_LEN, DIM), dtype=np.float32)
    pos[:, 0::2] = np.sin(para)
    pos[:, 1::2] = np.cos(para)
    return pos


_MESH = plsc.VectorSubcoreMesh(core_axis_name="c", subcore_axis_name="s",
                               num_cores=NC, num_subcores=NS)


@functools.partial(
    pl.kernel,
    out_type=jax.ShapeDtypeStruct((SEQ, DT, NW, DR, BPW), jnp.float32),
    mesh=_MESH,
    compiler_params=pltpu.CompilerParams(use_tc_tiling_on_sc=False,
                                         needs_layout_passes=False),
    scratch_types=[
        pltpu.VMEM((SEQ, BPW), jnp.int32),
        pltpu.VMEM((SEQ, DIM), jnp.float32),
        [pltpu.VMEM((LPC * BPW, DIM), jnp.float32)] * 2,
        [pltpu.VMEM((LPC, DT, DR, BPW), jnp.float32)] * 2,
        [pltpu.SemaphoreType.DMA] * 2,
        [pltpu.SemaphoreType.DMA] * 2,
    ],
)
def _sc_embed(w_hbm, xt_hbm, pos_hbm, out_hbm, idx_all, pos_v, rows, trans,
              gsem, ssem):
    wid = lax.axis_index("s") * NC + lax.axis_index("c")
    pltpu.sync_copy(pos_hbm, pos_v)
    b0 = pl.multiple_of(wid * BPW, BPW)
    pltpu.sync_copy(xt_hbm.at[:, pl.ds(b0, BPW)], idx_all)

    def fire_gather(c, rb):
        for j in range(LPC):
            pltpu.async_copy(w_hbm.at[idx_all.at[c * LPC + j]],
                             rows[rb].at[pl.ds(j * BPW, BPW)], gsem[rb])

    def drain_gather(rb):
        pltpu.make_async_copy(w_hbm.at[pl.ds(0, LPC * BPW)], rows[rb],
                              gsem[rb]).wait()

    def wait_store(tb):
        pltpu.make_async_copy(trans[tb], out_hbm.at[pl.ds(0, LPC), :, 0],
                              ssem[tb]).wait()

    iot = lax.iota(jnp.int32, 16)
    dtv = lax.shift_right_logical(iot, 3)
    dtv2 = dtv + 2
    drv = lax.bitwise_and(iot, 7)

    def compute(c, rb, tb):
        rbuf, tbuf = rows[rb], trans[tb]
        for lr in range(LPC):
            l = c * LPC + lr
            p0 = pos_v[l, pl.ds(0, 16)]
            p1 = pos_v[l, pl.ds(16, 16)]
            lv = jnp.full((16,), lr, jnp.int32)

            def tok_body(bq, _, lr=lr, p0=p0, p1=p1, lv=lv):
                for bs in range(8):
                    b = bq * 8 + bs
                    bv = jnp.full((16,), b, jnp.int32)
                    r0 = rbuf[lr * BPW + b, pl.ds(0, 16)] * SCALE + p0
                    r1 = rbuf[lr * BPW + b, pl.ds(16, 16)] * SCALE + p1
                    plsc.store_scatter(tbuf, [lv, dtv, drv, bv], r0)
                    plsc.store_scatter(tbuf, [lv, dtv2, drv, bv], r1)
                return 0

            lax.fori_loop(0, BPW // 8, tok_body, 0)

    fire_gather(0, 0)

    def pair_body(p, _):
        for rb in range(2):
            c = p * 2 + rb

            @pl.when(c + 1 < CHUNKS)
            def _(c=c, rb=rb):
                fire_gather(c + 1, 1 - rb)

            drain_gather(rb)

            @pl.when(c >= 2)
            def _(rb=rb):
                wait_store(rb)

            compute(c, rb, rb)
            pltpu.async_copy(
                trans[rb],
                out_hbm.at[pl.ds(pl.multiple_of(c * LPC, LPC), LPC), :, wid],
                ssem[rb])
        return 0

    lax.fori_loop(0, CHUNKS // 2, pair_body, 0)
    wait_store(0)
    wait_store(1)


def kernel(x, W):
    pos = jnp.asarray(_make_pos_table())
    xt = x.T
    out5 = _sc_embed(W, xt, pos)
    return out5.transpose(2, 4, 0, 1, 3).reshape(BATCH, SEQ, DIM)


# speedup vs baseline: 1.4837x; 1.4837x over previous
import functools
import math

import jax
import jax.numpy as jnp
import numpy as np
from jax import lax
from jax.experimental import pallas as pl
from jax.experimental.pallas import tpu as pltpu
from jax.experimental.pallas import tpu_sc as plsc

VOCAB = 1000000
DIM = 32
MAX_LEN = 200
BATCH = 4096
SEQ = 200

NC, NS = 2, 16
NW = NC * NS
BPW = BATCH // NW
LPC = 4
CHUNKS = SEQ // LPC
DT, DR = DIM // 8, 8
SCALE = math.sqrt(DIM)


def _make_pos_table():
    para = np.arange(MAX_LEN).reshape(-1, 1) / np.power(
        10000.0, np.arange(0, DIM, 2) / DIM)
    pos = np.zeros((MAX_LEN, DIM), dtype=np.float32)
    pos[:, 0::2] = np.sin(para)
    pos[:, 1::2] = np.cos(para)
    return pos


_MESH = plsc.VectorSubcoreMesh(core_axis_name="c", subcore_axis_name="s",
                               num_cores=NC, num_subcores=NS)


@functools.partial(
    pl.kernel,
    out_type=jax.ShapeDtypeStruct((SEQ, DT, NW, DR, BPW), jnp.float32),
    mesh=_MESH,
    compiler_params=pltpu.CompilerParams(use_tc_tiling_on_sc=False,
                                         needs_layout_passes=False),
    scratch_types=[
        pltpu.VMEM((SEQ, BPW), jnp.int32),
        pltpu.VMEM((SEQ, DIM), jnp.float32),
        [pltpu.VMEM((LPC * BPW, DIM), jnp.float32)] * 2,
        [pltpu.VMEM((LPC, DT, DR, BPW + 1), jnp.float32)] * 2,
        [pltpu.SemaphoreType.DMA] * 2,
        [pltpu.SemaphoreType.DMA] * 2,
    ],
)
def _sc_embed(w_hbm, xt_hbm, pos_hbm, out_hbm, idx_all, pos_v, rows, trans,
              gsem, ssem):
    wid = lax.axis_index("s") * NC + lax.axis_index("c")
    pltpu.sync_copy(pos_hbm, pos_v)
    b0 = pl.multiple_of(wid * BPW, BPW)
    pltpu.sync_copy(xt_hbm.at[:, pl.ds(b0, BPW)], idx_all)

    def fire_gather(c, rb):
        for j in range(LPC):
            pltpu.async_copy(w_hbm.at[idx_all.at[c * LPC + j]],
                             rows[rb].at[pl.ds(j * BPW, BPW)], gsem[rb])

    def drain_gather(rb):
        pltpu.make_async_copy(w_hbm.at[pl.ds(0, LPC * BPW)], rows[rb],
                              gsem[rb]).wait()

    def wait_store(tb):
        pltpu.make_async_copy(trans[tb].at[:, :, :, pl.ds(0, BPW)],
                              out_hbm.at[pl.ds(0, LPC), :, 0],
                              ssem[tb]).wait()

    iot = lax.iota(jnp.int32, 16)
    dtv = lax.shift_right_logical(iot, 3)
    dtv2 = dtv + 2
    drv = lax.bitwise_and(iot, 7)

    def compute(c, rb, tb):
        rbuf, tbuf = rows[rb], trans[tb]
        for lr in range(LPC):
            l = c * LPC + lr
            p0 = pos_v[l, pl.ds(0, 16)]
            p1 = pos_v[l, pl.ds(16, 16)]
            lv = jnp.full((16,), lr, jnp.int32)

            def tok_body(bq, _, lr=lr, p0=p0, p1=p1, lv=lv):
                for bs in range(8):
                    b = bq * 8 + bs
                    bv = jnp.full((16,), b, jnp.int32)
                    r0 = rbuf[lr * BPW + b, pl.ds(0, 16)] * SCALE + p0
                    r1 = rbuf[lr * BPW + b, pl.ds(16, 16)] * SCALE + p1
                    plsc.store_scatter(tbuf, [lv, dtv, drv, bv], r0)
                    plsc.store_scatter(tbuf, [lv, dtv2, drv, bv], r1)
                return 0

            lax.fori_loop(0, BPW // 8, tok_body, 0)

    fire_gather(0, 0)

    def pair_body(p, _):
        for rb in range(2):
            c = p * 2 + rb

            @pl.when(c + 1 < CHUNKS)
            def _(c=c, rb=rb):
                fire_gather(c + 1, 1 - rb)

            drain_gather(rb)

            @pl.when(c >= 2)
            def _(rb=rb):
                wait_store(rb)

            compute(c, rb, rb)
            for lr in range(LPC):
                pltpu.async_copy(
                    trans[rb].at[lr, :, :, pl.ds(0, BPW)],
                    out_hbm.at[c * LPC + lr, :, wid],
                    ssem[rb])
        return 0

    lax.fori_loop(0, CHUNKS // 2, pair_body, 0)
    wait_store(0)
    wait_store(1)


def kernel(x, W):
    pos = jnp.asarray(_make_pos_table())
    xt = x.T
    out5 = _sc_embed(W, xt, pos)
    return out5.transpose(2, 4, 0, 1, 3).reshape(BATCH, SEQ, DIM)
